# baseline (speedup 1.0000x reference)
import jax
import jax.numpy as jnp
from jax import lax
from jax.experimental import pallas as pl
from jax.experimental.pallas import tpu as pltpu
from jax.experimental.pallas import tpu_sc as plsc

NUM_CORES = 2
NUM_SUBCORES = 16
NUM_LANES = 16
NW = NUM_CORES * NUM_SUBCORES

NUM_ROWS = 1000000
BATCH = 16384
EMB_DIM = 32
B_PER_W = BATCH // NW
IDX_CHUNK = 128
N_CHUNKS = B_PER_W // IDX_CHUNK
N_VECS = B_PER_W // NUM_LANES

ALIGNED_ROWS = 999936
TAIL_ROWS = NUM_ROWS - ALIGNED_ROWS
FLAT_BULK = EMB_DIM * ALIGNED_ROWS
FLAT_SIZE = FLAT_BULK + EMB_DIM * TAIL_ROWS
DETILE_CHUNK = 131072


def _mf_body(user_ref, item_ref, ut_ref, it_ref, out_ref,
             idx_u, idx_i, off, elems_u, elems_i, out_v, sem):
  wid = lax.axis_index("s") * NUM_CORES + lax.axis_index("c")
  base = wid * B_PER_W

  pltpu.sync_copy(user_ref.at[pl.ds(base, B_PER_W)], idx_u)
  pltpu.sync_copy(item_ref.at[pl.ds(base, B_PER_W)], idx_i)

  def for_each_j(j, table_ref, idx, elems):
    def build(v, carry):
      sl = pl.ds(v * NUM_LANES, NUM_LANES)
      r = idx[sl]
      bulk = r + j * ALIGNED_ROWS
      tail = (r << 5) + j
      off[sl] = jnp.where(r < ALIGNED_ROWS, bulk, tail)
      return carry
    lax.fori_loop(0, N_VECS, build, 0)
    for c in range(N_CHUNKS):
      sl = pl.ds(c * IDX_CHUNK, IDX_CHUNK)
      pltpu.async_copy(table_ref.at[off.at[sl]], elems.at[j, sl], sem)
    for c in range(N_CHUNKS):
      sl = pl.ds(c * IDX_CHUNK, IDX_CHUNK)
      pltpu.make_async_copy(table_ref.at[off.at[sl]], elems.at[j, sl],
                            sem).wait()

  for j in range(EMB_DIM):
    for_each_j(j, ut_ref, idx_u, elems_u)
    for_each_j(j, it_ref, idx_i, elems_i)

  def group(m, carry):
    sl = pl.ds(m * NUM_LANES, NUM_LANES)
    acc = elems_u[0, sl] * elems_i[0, sl]
    for j in range(1, EMB_DIM):
      acc = acc + elems_u[j, sl] * elems_i[j, sl]
    out_v[sl] = 1.0 / (1.0 + jnp.exp(-acc))
    return carry

  lax.fori_loop(0, N_VECS, group, 0)

  pltpu.sync_copy(out_v, out_ref.at[pl.ds(base, B_PER_W)])


def _detile_body(ut_ref, it_ref, utail_ref, itail_ref,
                 uflat_ref, iflat_ref, sem):
  def copies():
    for table, tail, flat in ((ut_ref, utail_ref, uflat_ref),
                              (it_ref, itail_ref, iflat_ref)):
      for j in range(EMB_DIM):
        for c in range(0, ALIGNED_ROWS, DETILE_CHUNK):
          size = min(DETILE_CHUNK, ALIGNED_ROWS - c)
          yield (table.at[j, pl.ds(c, size)],
                 flat.at[pl.ds(j * ALIGNED_ROWS + c, size)])
      yield (tail, flat.at[pl.ds(FLAT_BULK, EMB_DIM * TAIL_ROWS)])

  for src, dst in copies():
    pltpu.make_async_copy(src, dst, sem).start()
  for src, dst in copies():
    pltpu.make_async_copy(src, dst, sem).wait()


def _detile(ut, it, utail, itail):
  return pl.pallas_call(
      _detile_body,
      in_specs=[pl.BlockSpec(memory_space=pl.ANY)] * 4,
      out_specs=[pl.BlockSpec(memory_space=pl.ANY)] * 2,
      out_shape=[jax.ShapeDtypeStruct((FLAT_SIZE,), jnp.float32)] * 2,
      scratch_shapes=[pltpu.SemaphoreType.DMA],
  )(ut, it, utail, itail)


@jax.jit
def kernel(user, item, user_table, item_table):
  utail = user_table[ALIGNED_ROWS:].reshape(-1)
  itail = item_table[ALIGNED_ROWS:].reshape(-1)
  ut, it = _detile(user_table.T, item_table.T, utail, itail)
  mesh = plsc.VectorSubcoreMesh(core_axis_name="c", subcore_axis_name="s")
  run = pl.kernel(
      _mf_body,
      out_type=jax.ShapeDtypeStruct((BATCH,), jnp.float32),
      mesh=mesh,
      compiler_params=pltpu.CompilerParams(
          needs_layout_passes=False,
          use_tc_tiling_on_sc=False,
      ),
      scratch_types=[
          pltpu.VMEM((B_PER_W,), jnp.int32),
          pltpu.VMEM((B_PER_W,), jnp.int32),
          pltpu.VMEM((B_PER_W,), jnp.int32),
          pltpu.VMEM((EMB_DIM, B_PER_W), jnp.float32),
          pltpu.VMEM((EMB_DIM, B_PER_W), jnp.float32),
          pltpu.VMEM((B_PER_W,), jnp.float32),
          pltpu.SemaphoreType.DMA,
      ],
  )
  return run(user, item, ut, it)
